# baseline (speedup 1.0000x reference)
import functools

import jax
import jax.numpy as jnp
from jax import lax
from jax.experimental import pallas as pl
from jax.experimental.pallas import tpu as pltpu
from jax.experimental.pallas import tpu_sc as plsc

TOP_K = 10


def _topk_body(y_ref, ts_ref, idx_ref, sel_ref):
    B, K = y_ref.shape
    colf = lax.broadcasted_iota(jnp.int32, (B, K), 1).astype(jnp.float32)
    row_base = lax.broadcasted_iota(jnp.int32, (B, 1), 0) * K

    nseg = 2
    seg = K // nseg
    v_cols, i_cols, t_cols = [], [], []
    for s in range(nseg):
        sl = slice(s * seg, (s + 1) * seg)
        cur = y_ref[:, sl]
        ts_s = ts_ref[:, sl]
        colf_s = colf[:, sl]
        for _ in range(TOP_K):
            m = jnp.max(cur, axis=1, keepdims=True)
            cand = jnp.where(cur == m, colf_s, float(K))
            idxf = jnp.min(cand, axis=1, keepdims=True)
            onehot = cand == idxf
            tk = jnp.sum(jnp.where(onehot, ts_s, 0.0), axis=1, keepdims=True)
            v_cols.append(m)
            i_cols.append(idxf)
            t_cols.append(tk)
            cur = jnp.where(onehot, -jnp.inf, cur)

    vals = jnp.concatenate(v_cols, axis=1)
    idxs = jnp.concatenate(i_cols, axis=1)
    tss = jnp.concatenate(t_cols, axis=1)

    idx_cols, tsel_cols, ysel_cols = [], [], []
    for _ in range(TOP_K):
        m = jnp.max(vals, axis=1, keepdims=True)
        cand = jnp.where(vals == m, idxs, float(K))
        idxf = jnp.min(cand, axis=1, keepdims=True)
        onehot = cand == idxf
        tk = jnp.sum(jnp.where(onehot, tss, 0.0), axis=1, keepdims=True)
        idx_cols.append(idxf.astype(jnp.int32) + row_base)
        tsel_cols.append(tk)
        ysel_cols.append(m)
        vals = jnp.where(onehot, -jnp.inf, vals)

    idx_ref[...] = jnp.concatenate(idx_cols, axis=1).T
    sel_ref[...] = jnp.concatenate(tsel_cols + ysel_cols, axis=1)


def _topk_stage(y, ts):
    B, K = y.shape
    return pl.pallas_call(
        _topk_body,
        out_shape=(
            jax.ShapeDtypeStruct((TOP_K, B), jnp.int32),
            jax.ShapeDtypeStruct((B, 2 * TOP_K), jnp.float32),
        ),
    )(y, ts)


def _sc_gather(table, idx2d):
    info = plsc.get_sparse_core_info()
    nc = 1
    kp, b = idx2d.shape
    d = table.shape[1]
    per = b
    mesh = plsc.VectorSubcoreMesh(core_axis_name="c", subcore_axis_name="s",
                                  num_cores=nc)

    hp = per // 2

    @functools.partial(
        pl.kernel,
        mesh=mesh,
        out_type=jax.ShapeDtypeStruct((kp * b, d), jnp.float32),
        scratch_types=[
            pltpu.VMEM((per,), jnp.int32),
            pltpu.VMEM((hp, d), jnp.float32),
            pltpu.VMEM((hp, d), jnp.float32),
            pltpu.SemaphoreType.DMA,
            pltpu.SemaphoreType.DMA,
            pltpu.SemaphoreType.DMA,
        ],
    )
    def gather_k(table_hbm, idx_hbm, out_hbm, idx_v, rows0, rows1, s0, s1, s2):
        wid = lax.axis_index("s") * nc + lax.axis_index("c")

        @pl.when(wid < kp)
        def _():
            base = wid * per
            pltpu.sync_copy(idx_hbm.at[wid], idx_v)
            g0 = pltpu.async_copy(table_hbm.at[idx_v.at[pl.ds(0, hp)]],
                                  rows0, s0)
            g1 = pltpu.async_copy(table_hbm.at[idx_v.at[pl.ds(hp, hp)]],
                                  rows1, s1)
            g0.wait()
            w0 = pltpu.async_copy(rows0, out_hbm.at[pl.ds(base, hp)], s2)
            g1.wait()
            w0.wait()
            pltpu.sync_copy(rows1, out_hbm.at[pl.ds(base + hp, hp)])

    return gather_k(table, idx2d)


def _mlp_body(g_ref, sel_ref, w1_ref, b1_ref, w2_ref, b2_ref, out_ref):
    B = sel_ref.shape[0]
    D = g_ref.shape[1]
    H = w2_ref.shape[1]
    g = g_ref[...]
    tsel = sel_ref[:, :TOP_K]
    ysel = sel_ref[:, TOP_K:]
    dot = functools.partial(
        lax.dot_general,
        dimension_numbers=(((1,), (0,)), ((), ())),
        preferred_element_type=jnp.float32,
        precision=lax.Precision.DEFAULT,
    )
    a = dot(g, w1_ref[:D, :]) + b1_ref[...]
    bb = dot(g, w1_ref[D:2 * D, :])
    w1c = w1_ref[2 * D:2 * D + 1, :]
    b2 = b2_ref[0, 0]

    t_acc = jnp.zeros((B, H), jnp.float32)
    for i in range(TOP_K):
        a_i = a[i * B:(i + 1) * B, :]
        t_i = tsel[:, i:i + 1]
        y_i = ysel[:, i:i + 1]
        for j in range(i + 1, TOP_K):
            dt = jnp.abs(t_i - tsel[:, j:j + 1])
            h = a_i + bb[j * B:(j + 1) * B, :] + dt * w1c
            s = h * jax.nn.sigmoid(h)
            t_acc = t_acc + s * (y_i * ysel[:, j:j + 1])
    ys = jnp.sum(ysel, axis=1, keepdims=True)
    w_sum = 0.5 * (ys * ys - jnp.sum(ysel * ysel, axis=1, keepdims=True))
    e = jnp.sum(t_acc * w2_ref[...], axis=1, keepdims=True) + b2 * w_sum
    out_ref[...] = e[:, 0]


def _mlp_stage(g, sel, w1, b1, w2, b2):
    B = sel.shape[0]
    return pl.pallas_call(
        _mlp_body,
        out_shape=jax.ShapeDtypeStruct((B,), jnp.float32),
    )(g, sel, w1, b1, w2, b2)


def kernel(M, y, timestamps, W1, b1, W2, b2):
    B, K, D = M.shape
    H = W1.shape[1]

    idx, sel = _topk_stage(y, timestamps)

    g = _sc_gather(M.reshape(B * K, D), idx)

    return _mlp_stage(g, sel, W1, b1.reshape(1, H),
                      W2.reshape(1, H), b2.reshape(1, 1))


# speedup vs baseline: 1.0559x; 1.0559x over previous
import functools

import jax
import jax.numpy as jnp
from jax import lax
from jax.experimental import pallas as pl
from jax.experimental.pallas import tpu as pltpu
from jax.experimental.pallas import tpu_sc as plsc

TOP_K = 10


def _topk_body(y_ref, ts_ref, idx_ref, sel_ref):
    B, K = y_ref.shape
    y = y_ref[...]
    ts = ts_ref[...]
    colf = lax.broadcasted_iota(jnp.int32, (B, K), 1).astype(jnp.float32)
    row_base = lax.broadcasted_iota(jnp.int32, (B, 1), 0) * K

    idx_cols = []
    tsel_cols = []
    ysel_cols = []
    cur = y
    for _ in range(TOP_K):
        m = jnp.max(cur, axis=1, keepdims=True)
        cand = jnp.where(cur == m, colf, float(K))
        idxf = jnp.min(cand, axis=1, keepdims=True)
        onehot = cand == idxf
        tk = jnp.sum(jnp.where(onehot, ts, 0.0), axis=1, keepdims=True)
        idx_cols.append(idxf.astype(jnp.int32) + row_base)
        tsel_cols.append(tk)
        ysel_cols.append(m)
        cur = jnp.where(onehot, -jnp.inf, cur)

    idx_ref[...] = jnp.concatenate(idx_cols, axis=1).T
    sel_ref[...] = jnp.concatenate(tsel_cols + ysel_cols, axis=1)


def _topk_stage(y, ts):
    B, K = y.shape
    return pl.pallas_call(
        _topk_body,
        out_shape=(
            jax.ShapeDtypeStruct((TOP_K, B), jnp.int32),
            jax.ShapeDtypeStruct((B, 2 * TOP_K), jnp.float32),
        ),
    )(y, ts)


def _sc_gather(table, idx2d):
    info = plsc.get_sparse_core_info()
    nc = 1
    kp, b = idx2d.shape
    d = table.shape[1]
    per = b
    mesh = plsc.VectorSubcoreMesh(core_axis_name="c", subcore_axis_name="s",
                                  num_cores=nc)

    hp = per // 2

    @functools.partial(
        pl.kernel,
        mesh=mesh,
        out_type=jax.ShapeDtypeStruct((kp * b, d), jnp.float32),
        scratch_types=[
            pltpu.VMEM((per,), jnp.int32),
            pltpu.VMEM((hp, d), jnp.float32),
            pltpu.VMEM((hp, d), jnp.float32),
            pltpu.SemaphoreType.DMA,
            pltpu.SemaphoreType.DMA,
            pltpu.SemaphoreType.DMA,
        ],
    )
    def gather_k(table_hbm, idx_hbm, out_hbm, idx_v, rows0, rows1, s0, s1, s2):
        wid = lax.axis_index("s") * nc + lax.axis_index("c")

        @pl.when(wid < kp)
        def _():
            base = wid * per
            pltpu.sync_copy(idx_hbm.at[wid], idx_v)
            g0 = pltpu.async_copy(table_hbm.at[idx_v.at[pl.ds(0, hp)]],
                                  rows0, s0)
            g1 = pltpu.async_copy(table_hbm.at[idx_v.at[pl.ds(hp, hp)]],
                                  rows1, s1)
            g0.wait()
            w0 = pltpu.async_copy(rows0, out_hbm.at[pl.ds(base, hp)], s2)
            g1.wait()
            w0.wait()
            pltpu.sync_copy(rows1, out_hbm.at[pl.ds(base + hp, hp)])

    return gather_k(table, idx2d)


def _mlp_body(g_ref, sel_ref, w1_ref, b1_ref, w2_ref, b2_ref, out_ref):
    B = sel_ref.shape[0]
    D = g_ref.shape[1]
    H = w2_ref.shape[1]
    g = g_ref[...]
    tsel = sel_ref[:, :TOP_K]
    ysel = sel_ref[:, TOP_K:]
    dot = functools.partial(
        lax.dot_general,
        dimension_numbers=(((1,), (0,)), ((), ())),
        preferred_element_type=jnp.float32,
        precision=lax.Precision.DEFAULT,
    )
    a = dot(g, w1_ref[:D, :]) + b1_ref[...]
    bb = dot(g, w1_ref[D:2 * D, :])
    w1c = w1_ref[2 * D:2 * D + 1, :]
    b2 = b2_ref[0, 0]

    ones_row = jnp.ones((1, H), jnp.float32)
    t_acc = jnp.zeros((B, H), jnp.float32)
    for i in range(TOP_K):
        a_i = a[i * B:(i + 1) * B, :]
        t_i = tsel[:, i:i + 1]
        y_i = ysel[:, i:i + 1]
        for j in range(i + 1, TOP_K):
            dt = jnp.abs(t_i - tsel[:, j:j + 1])
            h = a_i + bb[j * B:(j + 1) * B, :] + dot(dt, w1c)
            s = h * jax.nn.sigmoid(h)
            t_acc = t_acc + s * dot(y_i * ysel[:, j:j + 1], ones_row)
    ys = jnp.sum(ysel, axis=1, keepdims=True)
    w_sum = 0.5 * (ys * ys - jnp.sum(ysel * ysel, axis=1, keepdims=True))
    e = jnp.sum(t_acc * w2_ref[...], axis=1, keepdims=True) + b2 * w_sum
    out_ref[...] = e[:, 0]


def _mlp_stage(g, sel, w1, b1, w2, b2):
    B = sel.shape[0]
    return pl.pallas_call(
        _mlp_body,
        out_shape=jax.ShapeDtypeStruct((B,), jnp.float32),
    )(g, sel, w1, b1, w2, b2)


def kernel(M, y, timestamps, W1, b1, W2, b2):
    B, K, D = M.shape
    H = W1.shape[1]

    idx, sel = _topk_stage(y, timestamps)

    g = _sc_gather(M.reshape(B * K, D), idx)

    return _mlp_stage(g, sel, W1, b1.reshape(1, H),
                      W2.reshape(1, H), b2.reshape(1, 1))


# speedup vs baseline: 1.0728x; 1.0160x over previous
import functools

import jax
import jax.numpy as jnp
from jax import lax
from jax.experimental import pallas as pl
from jax.experimental.pallas import tpu as pltpu
from jax.experimental.pallas import tpu_sc as plsc

TOP_K = 10


def _topk_body(y_ref, ts_ref, idx_ref, sel_ref):
    B, K = y_ref.shape
    y = y_ref[...]
    ts = ts_ref[...]
    colf = lax.broadcasted_iota(jnp.int32, (B, K), 1).astype(jnp.float32)
    row_base = lax.broadcasted_iota(jnp.int32, (B, 1), 0) * K

    idx_cols = []
    tsel_cols = []
    ysel_cols = []
    cur = y
    for _ in range(TOP_K):
        m = jnp.max(cur, axis=1, keepdims=True)
        cand = jnp.where(cur == m, colf, float(K))
        idxf = jnp.min(cand, axis=1, keepdims=True)
        onehot = cand == idxf
        tk = jnp.sum(jnp.where(onehot, ts, 0.0), axis=1, keepdims=True)
        idx_cols.append(idxf.astype(jnp.int32) + row_base)
        tsel_cols.append(tk)
        ysel_cols.append(m)
        cur = jnp.where(onehot, -jnp.inf, cur)

    idx_ref[...] = jnp.concatenate(idx_cols, axis=1).T
    sel_ref[...] = jnp.concatenate(tsel_cols + ysel_cols, axis=1)


def _topk_stage(y, ts):
    B, K = y.shape
    return pl.pallas_call(
        _topk_body,
        out_shape=(
            jax.ShapeDtypeStruct((TOP_K, B), jnp.int32),
            jax.ShapeDtypeStruct((B, 2 * TOP_K), jnp.float32),
        ),
    )(y, ts)


def _sc_gather(table, idx2d):
    info = plsc.get_sparse_core_info()
    nc = 1
    kp, b = idx2d.shape
    d = table.shape[1]
    per = b
    mesh = plsc.VectorSubcoreMesh(core_axis_name="c", subcore_axis_name="s",
                                  num_cores=nc)

    nch = 4
    hp = per // nch

    @functools.partial(
        pl.kernel,
        mesh=mesh,
        out_type=jax.ShapeDtypeStruct((kp * b, d), jnp.float32),
        scratch_types=[
            pltpu.VMEM((per,), jnp.int32),
        ] + [pltpu.VMEM((hp, d), jnp.float32) for _ in range(nch)]
          + [pltpu.SemaphoreType.DMA, pltpu.SemaphoreType.DMA],
    )
    def gather_k(table_hbm, idx_hbm, out_hbm, idx_v, *bufs_sems):
        rows = bufs_sems[:nch]
        gsem, wsem = bufs_sems[nch], bufs_sems[nch + 1]
        wid = lax.axis_index("s") * nc + lax.axis_index("c")

        @pl.when(wid < kp)
        def _():
            base = wid * per
            pltpu.sync_copy(idx_hbm.at[wid], idx_v)
            gs = [pltpu.async_copy(table_hbm.at[idx_v.at[pl.ds(c * hp, hp)]],
                                   rows[c], gsem) for c in range(nch)]
            ws = []
            for c in range(nch):
                gs[c].wait()
                ws.append(pltpu.async_copy(
                    rows[c], out_hbm.at[pl.ds(base + c * hp, hp)], wsem))
            for w in ws:
                w.wait()

    return gather_k(table, idx2d)


def _mlp_body(g_ref, sel_ref, w1_ref, b1_ref, w2_ref, b2_ref, out_ref):
    B = sel_ref.shape[0]
    D = g_ref.shape[1]
    H = w2_ref.shape[1]
    g = g_ref[...]
    tsel = sel_ref[:, :TOP_K]
    ysel = sel_ref[:, TOP_K:]
    dot = functools.partial(
        lax.dot_general,
        dimension_numbers=(((1,), (0,)), ((), ())),
        preferred_element_type=jnp.float32,
        precision=lax.Precision.DEFAULT,
    )
    a = dot(g, w1_ref[:D, :]) + b1_ref[...]
    bb = dot(g, w1_ref[D:2 * D, :])
    w1c = w1_ref[2 * D:2 * D + 1, :]
    b2 = b2_ref[0, 0]

    ones_row = jnp.ones((1, H), jnp.float32)
    t_acc = jnp.zeros((B, H), jnp.float32)
    for i in range(TOP_K):
        a_i = a[i * B:(i + 1) * B, :]
        t_i = tsel[:, i:i + 1]
        y_i = ysel[:, i:i + 1]
        for j in range(i + 1, TOP_K):
            dt = jnp.abs(t_i - tsel[:, j:j + 1])
            h = a_i + bb[j * B:(j + 1) * B, :] + dot(dt, w1c)
            s = h * jax.nn.sigmoid(h)
            t_acc = t_acc + s * dot(y_i * ysel[:, j:j + 1], ones_row)
    ys = jnp.sum(ysel, axis=1, keepdims=True)
    w_sum = 0.5 * (ys * ys - jnp.sum(ysel * ysel, axis=1, keepdims=True))
    e = jnp.sum(t_acc * w2_ref[...], axis=1, keepdims=True) + b2 * w_sum
    out_ref[...] = e[:, 0]


def _mlp_stage(g, sel, w1, b1, w2, b2):
    B = sel.shape[0]
    return pl.pallas_call(
        _mlp_body,
        out_shape=jax.ShapeDtypeStruct((B,), jnp.float32),
    )(g, sel, w1, b1, w2, b2)


def kernel(M, y, timestamps, W1, b1, W2, b2):
    B, K, D = M.shape
    H = W1.shape[1]

    idx, sel = _topk_stage(y, timestamps)

    g = _sc_gather(M.reshape(B * K, D), idx)

    return _mlp_stage(g, sel, W1, b1.reshape(1, H),
                      W2.reshape(1, H), b2.reshape(1, 1))


# speedup vs baseline: 1.0997x; 1.0251x over previous
import functools

import jax
import jax.numpy as jnp
from jax import lax
from jax.experimental import pallas as pl
from jax.experimental.pallas import tpu as pltpu
from jax.experimental.pallas import tpu_sc as plsc

TOP_K = 10


def _topk_body(y_ref, ts_ref, idx_ref, sel_ref):
    B, K = y_ref.shape
    y = y_ref[...]
    ts = ts_ref[...]
    colf = lax.broadcasted_iota(jnp.int32, (B, K), 1).astype(jnp.float32)
    row_base = lax.broadcasted_iota(jnp.int32, (B, 1), 0) * K

    idx_cols = []
    tsel_cols = []
    ysel_cols = []
    cur = y
    for _ in range(TOP_K):
        m = jnp.max(cur, axis=1, keepdims=True)
        cand = jnp.where(cur == m, colf, float(K))
        idxf = jnp.min(cand, axis=1, keepdims=True)
        onehot = cand == idxf
        tk = jnp.sum(jnp.where(onehot, ts, 0.0), axis=1, keepdims=True)
        idx_cols.append(idxf.astype(jnp.int32) + row_base)
        tsel_cols.append(tk)
        ysel_cols.append(m)
        cur = jnp.where(onehot, -jnp.inf, cur)

    idx_t = jnp.concatenate(idx_cols, axis=1).T
    for k in range(TOP_K):
        idx_ref[pl.ds(k * B, B)] = idx_t[k]
    sel_ref[...] = jnp.concatenate(tsel_cols + ysel_cols, axis=1)


def _topk_stage(y, ts):
    B, K = y.shape
    return pl.pallas_call(
        _topk_body,
        out_shape=(
            jax.ShapeDtypeStruct((TOP_K * B,), jnp.int32),
            jax.ShapeDtypeStruct((B, 2 * TOP_K), jnp.float32),
        ),
    )(y, ts)


def _sc_gather(table, flat_idx):
    info = plsc.get_sparse_core_info()
    nc = 1
    nw = nc * info.num_subcores
    (n,) = flat_idx.shape
    d = table.shape[1]
    per = n // nw
    mesh = plsc.VectorSubcoreMesh(core_axis_name="c", subcore_axis_name="s",
                                  num_cores=nc)

    nch = 5
    hp = per // nch

    @functools.partial(
        pl.kernel,
        mesh=mesh,
        out_type=jax.ShapeDtypeStruct((n, d), jnp.float32),
        scratch_types=[
            pltpu.VMEM((per,), jnp.int32),
        ] + [pltpu.VMEM((hp, d), jnp.float32) for _ in range(nch)]
          + [pltpu.SemaphoreType.DMA, pltpu.SemaphoreType.DMA],
    )
    def gather_k(table_hbm, idx_hbm, out_hbm, idx_v, *bufs_sems):
        rows = bufs_sems[:nch]
        gsem, wsem = bufs_sems[nch], bufs_sems[nch + 1]
        wid = lax.axis_index("s") * nc + lax.axis_index("c")
        base = wid * per
        pltpu.sync_copy(idx_hbm.at[pl.ds(base, per)], idx_v)
        gs = [pltpu.async_copy(table_hbm.at[idx_v.at[pl.ds(c * hp, hp)]],
                               rows[c], gsem) for c in range(nch)]
        ws = []
        for c in range(nch):
            gs[c].wait()
            ws.append(pltpu.async_copy(
                rows[c], out_hbm.at[pl.ds(base + c * hp, hp)], wsem))
        for w in ws:
            w.wait()

    return gather_k(table, flat_idx)


def _mlp_body(g_ref, sel_ref, w1_ref, b1_ref, w2_ref, b2_ref, out_ref):
    B = sel_ref.shape[0]
    D = g_ref.shape[1]
    H = w2_ref.shape[1]
    g = g_ref[...]
    tsel = sel_ref[:, :TOP_K]
    ysel = sel_ref[:, TOP_K:]
    dot = functools.partial(
        lax.dot_general,
        dimension_numbers=(((1,), (0,)), ((), ())),
        preferred_element_type=jnp.float32,
        precision=lax.Precision.DEFAULT,
    )
    a = dot(g, w1_ref[:D, :]) + b1_ref[...]
    bb = dot(g, w1_ref[D:2 * D, :])
    w1c = w1_ref[2 * D:2 * D + 1, :]
    b2 = b2_ref[0, 0]

    ones_row = jnp.ones((1, H), jnp.float32)
    t_acc = jnp.zeros((B, H), jnp.float32)
    for i in range(TOP_K):
        a_i = a[i * B:(i + 1) * B, :]
        t_i = tsel[:, i:i + 1]
        y_i = ysel[:, i:i + 1]
        for j in range(i + 1, TOP_K):
            dt = jnp.abs(t_i - tsel[:, j:j + 1])
            h = a_i + bb[j * B:(j + 1) * B, :] + dot(dt, w1c)
            s = h * jax.nn.sigmoid(h)
            t_acc = t_acc + s * dot(y_i * ysel[:, j:j + 1], ones_row)
    ys = jnp.sum(ysel, axis=1, keepdims=True)
    w_sum = 0.5 * (ys * ys - jnp.sum(ysel * ysel, axis=1, keepdims=True))
    e = jnp.sum(t_acc * w2_ref[...], axis=1, keepdims=True) + b2 * w_sum
    out_ref[...] = e[:, 0]


def _mlp_stage(g, sel, w1, b1, w2, b2):
    B = sel.shape[0]
    return pl.pallas_call(
        _mlp_body,
        out_shape=jax.ShapeDtypeStruct((B,), jnp.float32),
    )(g, sel, w1, b1, w2, b2)


def kernel(M, y, timestamps, W1, b1, W2, b2):
    B, K, D = M.shape
    H = W1.shape[1]

    idx, sel = _topk_stage(y, timestamps)

    g = _sc_gather(M.reshape(B * K, D), idx)

    return _mlp_stage(g, sel, W1, b1.reshape(1, H),
                      W2.reshape(1, H), b2.reshape(1, 1))
